# baseline (speedup 1.0000x reference)
import jax
import jax.numpy as jnp
from jax import lax
from jax.experimental import pallas as pl
from jax.experimental.pallas import tpu as pltpu
from jax.experimental.pallas import tpu_sc as plsc

_B = 8
_C_AB = 6
_C_B = 3
_H = 512
_W = 512
_S = 64
_PRO = 62
_R = 64
_NEG = -3.4e38


def _sc_loc_body(score_map, rcl, score_v, out_v):
    wid = lax.axis_index("s") * 2 + lax.axis_index("c")
    iota = lax.iota(jnp.int32, 16)

    @pl.when(wid < _B)
    def _():
        b = wid
        pltpu.sync_copy(score_map.at[b, 0], score_v)

        def row_step(r, carry):
            rmax, ridx = carry
            for k in range(4):
                v = score_v[r, pl.ds(16 * k, 16)]
                if k == 3:
                    v = jnp.where(iota < _PRO - 48, v, jnp.float32(_NEG))
                cm = jnp.max(v)
                pos = jnp.min(jnp.where(v == cm, iota, 16))
                flat = r * _PRO + 16 * k + pos
                upd = cm > rmax
                rmax = jnp.where(upd, cm, rmax)
                ridx = jnp.where(upd, flat, ridx)
            return rmax, ridx

        rmax, ridx = lax.fori_loop(0, _PRO, row_step,
                                   (jnp.float32(_NEG), jnp.int32(0)))
        valid = rmax > 0.0
        rb = jnp.where(valid, ridx // _PRO, 0) * 6 + 70
        cb = jnp.where(valid, ridx % _PRO, 0) * 6 + 70
        out_v[...] = jnp.where(iota < 8, rb, cb)
        pltpu.sync_copy(out_v, rcl.at[b])


_sc_loc = pl.kernel(
    _sc_loc_body,
    out_type=jax.ShapeDtypeStruct((_B, 16), jnp.int32),
    mesh=plsc.VectorSubcoreMesh(core_axis_name="c", subcore_axis_name="s"),
    compiler_params=pltpu.CompilerParams(use_tc_tiling_on_sc=False,
                                         needs_layout_passes=False),
    scratch_types=[
        pltpu.VMEM((_S, _S), jnp.float32),
        pltpu.VMEM((16,), jnp.int32),
    ],
)


def _tc_crop_body(rcl_ref, rB_ref, fB_ref, rBr_ref, fBr_ref, sr, sf):
    t = pl.program_id(2)
    sr[pl.ds(t * 8, 8), :] = rB_ref[0, 0]
    sf[pl.ds(t * 8, 8), :] = fB_ref[0, 0]

    @pl.when(t == 8)
    def _():
        b = pl.program_id(0)
        rb = rcl_ref[b, 0]
        cb = rcl_ref[b, 8]
        roff = rb - (rb // 8) * 8

        def win(s):
            v = pltpu.roll(s[...], lax.rem(72 - roff, 72), axis=0)
            v = pltpu.roll(v, _W - cb, axis=1)
            return v[:_R, :_R]

        rBr_ref[0, 0] = win(sr)
        fBr_ref[0, 0] = win(sf)


_tc_crop = pl.pallas_call(
    _tc_crop_body,
    grid_spec=pltpu.PrefetchScalarGridSpec(
        num_scalar_prefetch=1,
        grid=(_B, _C_B, 9),
        in_specs=[
            pl.BlockSpec((1, 1, 8, _W),
                         lambda b, c, t, rcl: (b, c, rcl[b, 0] // 8 + t, 0)),
            pl.BlockSpec((1, 1, 8, _W),
                         lambda b, c, t, rcl: (b, c, rcl[b, 0] // 8 + t, 0)),
        ],
        out_specs=[
            pl.BlockSpec((1, 1, _R, _R), lambda b, c, t, rcl: (b, c, 0, 0)),
            pl.BlockSpec((1, 1, _R, _R), lambda b, c, t, rcl: (b, c, 0, 0)),
        ],
        scratch_shapes=[
            pltpu.VMEM((72, _W), jnp.float32),
            pltpu.VMEM((72, _W), jnp.float32),
        ],
    ),
    out_shape=(
        jax.ShapeDtypeStruct((_B, _C_B, _R, _R), jnp.float32),
        jax.ShapeDtypeStruct((_B, _C_B, _R, _R), jnp.float32),
    ),
    compiler_params=pltpu.CompilerParams(
        dimension_semantics=("arbitrary", "arbitrary", "arbitrary"),
    ),
)


def _tc_copy_body(rcl_ref, rAB, fAB, out, sxb, ftb, bsem, ssem, osem):
    big = []
    for b in range(_B):
        d = pltpu.make_async_copy(rAB.at[b], out.at[b], bsem.at[b])
        d.start()
        big.append(d)

    stripes = []
    for b in range(_B):
        rb = rcl_ref[b, 0]
        rb8 = pl.multiple_of((rb // 8) * 8, 8)
        d0 = pltpu.make_async_copy(rAB.at[b, :, pl.ds(rb8, 8), :],
                                   sxb.at[b], ssem.at[2 * b])
        d1 = pltpu.make_async_copy(fAB.at[b, :, pl.ds(rb8, 8), :],
                                   ftb.at[b], ssem.at[2 * b + 1])
        d0.start()
        d1.start()
        stripes.append((d0, d1))

    r8 = lax.broadcasted_iota(jnp.int32, (_C_AB, 8, _W), 1)
    c8 = lax.broadcasted_iota(jnp.int32, (_C_AB, 8, _W), 2)
    for b in range(_B):
        rb = rcl_ref[b, 0]
        cb = rcl_ref[b, 8]
        roff = rb - (rb // 8) * 8
        d0, d1 = stripes[b]
        d0.wait()
        d1.wait()
        m = (r8 == roff) & (c8 == cb)
        sxb[b] = jnp.where(m, ftb[b], sxb[b])

    for d in big:
        d.wait()
    outs = []
    for b in range(_B):
        rb = rcl_ref[b, 0]
        rb8 = pl.multiple_of((rb // 8) * 8, 8)
        d = pltpu.make_async_copy(sxb.at[b],
                                  out.at[b, :, pl.ds(rb8, 8), :], osem.at[b])
        d.start()
        outs.append(d)
    for d in outs:
        d.wait()


_tc_copy = pl.pallas_call(
    _tc_copy_body,
    in_specs=[
        pl.BlockSpec(memory_space=pltpu.SMEM),
        pl.BlockSpec(memory_space=pltpu.HBM),
        pl.BlockSpec(memory_space=pltpu.HBM),
    ],
    out_specs=pl.BlockSpec(memory_space=pltpu.HBM),
    out_shape=jax.ShapeDtypeStruct((_B, _C_AB, _H, _W), jnp.float32),
    scratch_shapes=[
        pltpu.VMEM((_B, _C_AB, 8, _W), jnp.float32),
        pltpu.VMEM((_B, _C_AB, 8, _W), jnp.float32),
        pltpu.SemaphoreType.DMA((_B,)),
        pltpu.SemaphoreType.DMA((2 * _B,)),
        pltpu.SemaphoreType.DMA((_B,)),
    ],
)


def kernel(real_AB, fake_AB, score_map, real_B, fake_B):
    rcl = _sc_loc(score_map)
    real_Br, fake_Br = _tc_crop(rcl, real_B, fake_B)
    fake_ABm = _tc_copy(rcl, real_AB, fake_AB)
    return fake_ABm, real_Br, fake_Br
